# baseline (speedup 1.0000x reference)
import functools

import jax
import jax.numpy as jnp
from jax import lax
from jax.experimental import pallas as pl
from jax.experimental.pallas import tpu as pltpu
from jax.experimental.pallas import tpu_sc as plsc

NC = 2
NS = 16
NW = NC * NS
EC = 1600
ZC = 1000
BN = 1600


def _mesh():
    return plsc.VectorSubcoreMesh(core_axis_name="c", subcore_axis_name="s")


_SC_PARAMS = pltpu.CompilerParams(use_tc_tiling_on_sc=False)


def _sc_gather_pair(table_s, idx_s, table_d, idx_d):
    E = idx_s.shape[0]
    assert E % EC == 0
    nch = E // EC
    kpg = (nch + NW - 1) // NW
    f32 = jnp.float32

    @functools.partial(
        pl.kernel,
        out_type=(jax.ShapeDtypeStruct((E, 16), f32),
                  jax.ShapeDtypeStruct((E, 16), f32)),
        mesh=_mesh(),
        scratch_types=[
            pltpu.VMEM((EC,), jnp.int32), pltpu.VMEM((EC, 16), f32),
            pltpu.VMEM((EC,), jnp.int32), pltpu.VMEM((EC, 16), f32),
            pltpu.SemaphoreType.DMA, pltpu.SemaphoreType.DMA,
        ],
        compiler_params=_SC_PARAMS,
    )
    def k(ts, isr, td, idr, o_s, o_d, iv1, rv1, iv2, rv2, sem1, sem2):
        wid = lax.axis_index("s") * NC + lax.axis_index("c")

        def step(j, carry):
            c = wid + j * NW

            @pl.when(c < nch)
            def _():
                base = c * EC
                pltpu.sync_copy(isr.at[pl.ds(base, EC)], iv1)
                cp1 = pltpu.async_copy(ts.at[iv1], rv1, sem1)
                pltpu.sync_copy(idr.at[pl.ds(base, EC)], iv2)
                cp2 = pltpu.async_copy(td.at[iv2], rv2, sem2)
                cp1.wait()
                pltpu.sync_copy(rv1, o_s.at[pl.ds(base, EC)])
                cp2.wait()
                pltpu.sync_copy(rv2, o_d.at[pl.ds(base, EC)])

            return carry

        lax.fori_loop(0, kpg, step, 0)

    return k(table_s, idx_s, table_d, idx_d)


def _sc_scatter2(e1, i1, e2, i2, nseg):
    E1, E2 = i1.shape[0], i2.shape[0]
    assert E1 % EC == 0 and E2 % EC == 0 and nseg % ZC == 0
    n1, n2 = E1 // EC, E2 // EC
    nz = nseg // ZC
    f32 = jnp.float32
    zeros = jnp.zeros((ZC, 16), f32)

    @functools.partial(
        pl.kernel,
        out_type=jax.ShapeDtypeStruct((NC, nseg, 16), f32),
        mesh=_mesh(),
        scratch_types=[
            pltpu.VMEM((EC,), jnp.int32), pltpu.VMEM((EC, 16), f32),
            pltpu.VMEM_SHARED((nseg, 16), f32),
        ],
        compiler_params=_SC_PARAMS,
    )
    def k(e1r, i1r, e2r, i2r, zr, out, iv, vv, acc):
        cid = lax.axis_index("c")
        sid = lax.axis_index("s")
        wid = sid * NC + cid

        def zch(t, carry):
            z = sid + t * NS

            @pl.when(z < nz)
            def _():
                pltpu.sync_copy(zr, acc.at[pl.ds(z * ZC, ZC)])

            return carry

        lax.fori_loop(0, (nz + NS - 1) // NS, zch, 0)
        plsc.subcore_barrier()

        def scat(er, ir, nch):
            def step(j, carry):
                c = wid + j * NW

                @pl.when(c < nch)
                def _():
                    pltpu.sync_copy(ir.at[pl.ds(c * EC, EC)], iv)
                    pltpu.sync_copy(er.at[pl.ds(c * EC, EC)], vv)
                    pltpu.sync_copy(vv, acc.at[iv], add=True)

                return carry

            lax.fori_loop(0, (nch + NW - 1) // NW, step, 0)

        scat(e1r, i1r, n1)
        scat(e2r, i2r, n2)
        plsc.subcore_barrier()

        def sch(t, carry):
            z = sid + t * NS

            @pl.when(z < nz)
            def _():
                pltpu.sync_copy(acc.at[pl.ds(z * ZC, ZC)],
                                out.at[cid, pl.ds(z * ZC, ZC)])

            return carry

        lax.fori_loop(0, (nz + NS - 1) // NS, sch, 0)

    return k(e1, i1, e2, i2, zeros)


P = 8
BP = 800


def _bd(W):
    return jnp.kron(jnp.eye(P, dtype=W.dtype), W)


def _tc_mlp(xs, groups, p, ln, res_idx):
    M = xs[0].shape[0] if xs[0].ndim == 2 else xs[0].shape[1]
    H = p['Win'].shape[1]
    O = p['Wout'].shape[1]
    f32 = jnp.float32

    weights = []
    off = 0
    for g in groups:
        f = xs[g[0]].shape[-1] // P
        weights.append(_bd(p['Win'][off:off + f, :]))
        off += f
    weights.append(jnp.tile(p['bin'], P).reshape(1, P * H))
    for W, b in p['hidden']:
        weights += [_bd(W), jnp.tile(b, P).reshape(1, P * H)]
    weights += [_bd(p['Wout']), jnp.tile(p['bout'], P).reshape(1, P * O)]
    if ln:
        weights += [
            _bd(jnp.full((O, O), 1.0 / O, f32)),
            jnp.tile(p['gamma'], P).reshape(1, P * O),
            jnp.tile(p['beta'], P).reshape(1, P * O),
        ]
    nh = len(p['hidden'])

    def body(*refs):
        x_refs = refs[:len(xs)]
        w = refs[len(xs):-1]
        o_ref = refs[-1]
        xv = []
        for xr in x_refs:
            v = xr[...]
            if v.ndim == 3:
                v = v[0] + v[1]
            xv.append(v)
        a = None
        for gi_, g in enumerate(groups):
            xg = xv[g[0]]
            for gj in g[1:]:
                xg = xg + xv[gj]
            part = jnp.dot(xg, w[gi_][...], preferred_element_type=f32)
            a = part if a is None else a + part
        wi = len(groups)
        a = a + w[wi][...]
        wi += 1
        h = jnp.maximum(a, a * 0.01)
        for _ in range(nh):
            a2 = jnp.dot(h, w[wi][...], preferred_element_type=f32) + w[wi + 1][...]
            h = jnp.maximum(a2, a2 * 0.01)
            wi += 2
        y = jnp.dot(h, w[wi][...], preferred_element_type=f32) + w[wi + 1][...]
        wi += 2
        if ln:
            bdm = w[wi][...]
            hi = jax.lax.Precision.HIGHEST
            mu = jnp.dot(y, bdm, preferred_element_type=f32, precision=hi)
            d = y - mu
            var = jnp.dot(d * d, bdm, preferred_element_type=f32, precision=hi)
            y = d / jnp.sqrt(var + 1e-5) * w[wi + 1][...] + w[wi + 2][...]
        if res_idx is not None:
            y = y + xv[res_idx]
        o_ref[...] = y

    in_specs = []
    for x in xs:
        if x.ndim == 3:
            in_specs.append(pl.BlockSpec((NC, BP, x.shape[-1]),
                                         lambda i: (0, i, 0)))
        else:
            in_specs.append(pl.BlockSpec((BP, x.shape[-1]), lambda i: (i, 0)))
    for wt in weights:
        in_specs.append(pl.BlockSpec(wt.shape, lambda i: (0, 0)))

    return pl.pallas_call(
        body,
        grid=(pl.cdiv(M, BP),),
        in_specs=in_specs,
        out_specs=pl.BlockSpec((BP, P * O), lambda i: (i, 0)),
        out_shape=jax.ShapeDtypeStruct((M, P * O), f32),
    )(*xs, *weights)


def kernel(branch_x, junction_x, b2b_eattr, j2j_eattr, b2j_eattr, j2b_eattr,
           inlet_branch, outlet_branch, inlet_junction, outlet_junction, params,
           b2b_edges, j2j_edges, b2j_edges, j2b_edges):
    nb = branch_x.shape[0]
    nj = junction_x.shape[0]
    pk = lambda x: x.reshape(x.shape[0] // P, P * x.shape[1])
    pkc = lambda x: jnp.concatenate([x[j::P] for j in range(P)], axis=1)
    enc = lambda p, x: _tc_mlp([pkc(x)], [[0]], p, True, None)

    node_b = enc(params['enc_branch'], branch_x)
    node_j = enc(params['enc_junction'], junction_x)
    e_b2b = enc(params['enc_b2b'], b2b_eattr)
    e_j2j = enc(params['enc_j2j'], j2j_eattr)
    e_b2j = enc(params['enc_b2j'], b2j_eattr)
    e_j2b = enc(params['enc_b2j'], j2b_eattr)
    in_b = enc(params['enc_inlet'], inlet_branch)
    out_b = enc(params['enc_outlet'], outlet_branch)
    in_j = enc(params['enc_inlet'], inlet_junction)
    out_j = enc(params['enc_outlet'], outlet_junction)

    for i in range(3):
        tb = node_b.reshape(nb, 16)
        tj = node_j.reshape(nj, 16)
        gs, gd = _sc_gather_pair(tb, b2b_edges[0], tb, b2b_edges[1])
        e_b2b = _tc_mlp([e_b2b, pk(gs), pk(gd)], [[0], [1], [2]],
                        params['proc_b2b'][i], True, 0)
        gs, gd = _sc_gather_pair(tj, j2j_edges[0], tj, j2j_edges[1])
        e_j2j = _tc_mlp([e_j2j, pk(gs), pk(gd)], [[0], [1], [2]],
                        params['proc_j2j'][i], True, 0)
        gs, gd = _sc_gather_pair(tb, b2j_edges[0], tj, b2j_edges[1])
        e_b2j = _tc_mlp([e_b2j, pk(gs), pk(gd)], [[0], [1], [2]],
                        params['proc_b2j'][i], True, 0)
        gs, gd = _sc_gather_pair(tj, j2b_edges[0], tb, j2b_edges[1])
        e_j2b = _tc_mlp([e_j2b, pk(gs), pk(gd)], [[0], [1], [2]],
                        params['proc_j2b'][i], True, 0)
        pe_b = _sc_scatter2(e_b2b.reshape(-1, 16), b2b_edges[1],
                            e_j2b.reshape(-1, 16), j2b_edges[1], nb)
        pe_j = _sc_scatter2(e_j2j.reshape(-1, 16), j2j_edges[1],
                            e_b2j.reshape(-1, 16), b2j_edges[1], nj)
        pe_b = pe_b.reshape(NC, nb // P, P * 16)
        pe_j = pe_j.reshape(NC, nj // P, P * 16)
        node_b = _tc_mlp([node_b, pe_b, in_b, out_b], [[0], [1], [2], [3]],
                         params['proc_branch'][i], True, 0)
        node_j = _tc_mlp([node_j, pe_j, in_j, out_j], [[0], [1], [2], [3]],
                         params['proc_junction'][i], True, 0)

    h_b = _tc_mlp([node_b], [[0]], params['out_branch'], False, None)
    h_j = _tc_mlp([node_j], [[0]], params['out_junction'], False, None)
    return jnp.concatenate([h_b.reshape(nb, 2), h_j.reshape(nj, 2)], axis=0)


# speedup vs baseline: 1.2950x; 1.2950x over previous
import functools

import jax
import jax.numpy as jnp
from jax import lax
from jax.experimental import pallas as pl
from jax.experimental.pallas import tpu as pltpu
from jax.experimental.pallas import tpu_sc as plsc

NC = 2
NS = 16
NW = NC * NS
EC = 1600
ZC = 1000
BN = 1600


def _mesh():
    return plsc.VectorSubcoreMesh(core_axis_name="c", subcore_axis_name="s")


_SC_PARAMS = pltpu.CompilerParams(use_tc_tiling_on_sc=False)


def _sc_gather_pair(table_s, idx_s, table_d, idx_d):
    E = idx_s.shape[0]
    assert E % EC == 0
    nch = E // EC
    kpg = (nch + NW - 1) // NW
    f32 = jnp.float32

    @functools.partial(
        pl.kernel,
        out_type=(jax.ShapeDtypeStruct((E, 16), f32),
                  jax.ShapeDtypeStruct((E, 16), f32)),
        mesh=_mesh(),
        scratch_types=[
            pltpu.VMEM((EC,), jnp.int32), pltpu.VMEM((EC, 16), f32),
            pltpu.VMEM((EC,), jnp.int32), pltpu.VMEM((EC, 16), f32),
            pltpu.SemaphoreType.DMA, pltpu.SemaphoreType.DMA,
        ],
        compiler_params=_SC_PARAMS,
    )
    def k(ts, isr, td, idr, o_s, o_d, iv1, rv1, iv2, rv2, sem1, sem2):
        wid = lax.axis_index("s") * NC + lax.axis_index("c")

        def step(j, carry):
            c = wid + j * NW

            @pl.when(c < nch)
            def _():
                base = c * EC
                pltpu.sync_copy(isr.at[pl.ds(base, EC)], iv1)
                cp1 = pltpu.async_copy(ts.at[iv1], rv1, sem1)
                pltpu.sync_copy(idr.at[pl.ds(base, EC)], iv2)
                cp2 = pltpu.async_copy(td.at[iv2], rv2, sem2)
                cp1.wait()
                pltpu.sync_copy(rv1, o_s.at[pl.ds(base, EC)])
                cp2.wait()
                pltpu.sync_copy(rv2, o_d.at[pl.ds(base, EC)])

            return carry

        lax.fori_loop(0, kpg, step, 0)

    return k(table_s, idx_s, table_d, idx_d)


def _sc_scatter2(e1, i1, e2, i2, nseg):
    E1, E2 = i1.shape[0], i2.shape[0]
    assert E1 % EC == 0 and E2 % EC == 0 and nseg % ZC == 0
    n1, n2 = E1 // EC, E2 // EC
    nz = nseg // ZC
    f32 = jnp.float32
    zeros = jnp.zeros((ZC, 16), f32)

    @functools.partial(
        pl.kernel,
        out_type=jax.ShapeDtypeStruct((NC, nseg, 16), f32),
        mesh=_mesh(),
        scratch_types=[
            pltpu.VMEM((EC,), jnp.int32), pltpu.VMEM((EC, 16), f32),
            pltpu.VMEM_SHARED((nseg, 16), f32),
        ],
        compiler_params=_SC_PARAMS,
    )
    def k(e1r, i1r, e2r, i2r, zr, out, iv, vv, acc):
        cid = lax.axis_index("c")
        sid = lax.axis_index("s")
        wid = sid * NC + cid

        def zch(t, carry):
            z = sid + t * NS

            @pl.when(z < nz)
            def _():
                pltpu.sync_copy(zr, acc.at[pl.ds(z * ZC, ZC)])

            return carry

        lax.fori_loop(0, (nz + NS - 1) // NS, zch, 0)
        plsc.subcore_barrier()

        def scat(er, ir, nch):
            def step(j, carry):
                c = wid + j * NW

                @pl.when(c < nch)
                def _():
                    pltpu.sync_copy(ir.at[pl.ds(c * EC, EC)], iv)
                    pltpu.sync_copy(er.at[pl.ds(c * EC, EC)], vv)
                    pltpu.sync_copy(vv, acc.at[iv], add=True)

                return carry

            lax.fori_loop(0, (nch + NW - 1) // NW, step, 0)

        scat(e1r, i1r, n1)
        scat(e2r, i2r, n2)
        plsc.subcore_barrier()

        def sch(t, carry):
            z = sid + t * NS

            @pl.when(z < nz)
            def _():
                pltpu.sync_copy(acc.at[pl.ds(z * ZC, ZC)],
                                out.at[cid, pl.ds(z * ZC, ZC)])

            return carry

        lax.fori_loop(0, (nz + NS - 1) // NS, sch, 0)

    return k(e1, i1, e2, i2, zeros)


P = 8
BP = 800


def _bd(W):
    return jnp.kron(jnp.eye(P, dtype=W.dtype), W)


def _tc_mlp(xs, groups, p, ln, res_idx):
    M = xs[0].shape[0] if xs[0].ndim == 2 else xs[0].shape[1]
    H = p['Win'].shape[1]
    O = p['Wout'].shape[1]
    f32 = jnp.float32

    weights = []
    off = 0
    for g in groups:
        f = xs[g[0]].shape[-1] // P
        weights.append(_bd(p['Win'][off:off + f, :]))
        off += f
    weights.append(jnp.tile(p['bin'], P).reshape(1, P * H))
    for W, b in p['hidden']:
        weights += [_bd(W), jnp.tile(b, P).reshape(1, P * H)]
    wout, bout = p['Wout'], p['bout']
    if ln:
        wout = wout - jnp.mean(wout, axis=1, keepdims=True)
        bout = bout - jnp.mean(bout)
    weights += [_bd(wout), jnp.tile(bout, P).reshape(1, P * O)]
    if ln:
        weights += [
            _bd(jnp.full((O, O), 1.0 / O, f32)),
            jnp.tile(p['gamma'], P).reshape(1, P * O),
            jnp.tile(p['beta'], P).reshape(1, P * O),
        ]
    nh = len(p['hidden'])

    def body(*refs):
        x_refs = refs[:len(xs)]
        w = refs[len(xs):-1]
        o_ref = refs[-1]
        xv = []
        for xr in x_refs:
            v = xr[...]
            if v.ndim == 3:
                v = v[0] + v[1]
            xv.append(v)
        a = None
        for gi_, g in enumerate(groups):
            xg = xv[g[0]]
            for gj in g[1:]:
                xg = xg + xv[gj]
            part = jnp.dot(xg, w[gi_][...], preferred_element_type=f32)
            a = part if a is None else a + part
        wi = len(groups)
        a = a + w[wi][...]
        wi += 1
        h = jnp.maximum(a, a * 0.01)
        for _ in range(nh):
            a2 = jnp.dot(h, w[wi][...], preferred_element_type=f32) + w[wi + 1][...]
            h = jnp.maximum(a2, a2 * 0.01)
            wi += 2
        y = jnp.dot(h, w[wi][...], preferred_element_type=f32) + w[wi + 1][...]
        wi += 2
        if ln:
            bdm = w[wi][...]
            d = y
            var = jnp.dot(d * d, bdm, preferred_element_type=f32)
            y = d / jnp.sqrt(var + 1e-5) * w[wi + 1][...] + w[wi + 2][...]
        if res_idx is not None:
            y = y + xv[res_idx]
        o_ref[...] = y

    in_specs = []
    for x in xs:
        if x.ndim == 3:
            in_specs.append(pl.BlockSpec((NC, BP, x.shape[-1]),
                                         lambda i: (0, i, 0)))
        else:
            in_specs.append(pl.BlockSpec((BP, x.shape[-1]), lambda i: (i, 0)))
    for wt in weights:
        in_specs.append(pl.BlockSpec(wt.shape, lambda i: (0, 0)))

    return pl.pallas_call(
        body,
        grid=(pl.cdiv(M, BP),),
        in_specs=in_specs,
        out_specs=pl.BlockSpec((BP, P * O), lambda i: (i, 0)),
        out_shape=jax.ShapeDtypeStruct((M, P * O), f32),
    )(*xs, *weights)


def kernel(branch_x, junction_x, b2b_eattr, j2j_eattr, b2j_eattr, j2b_eattr,
           inlet_branch, outlet_branch, inlet_junction, outlet_junction, params,
           b2b_edges, j2j_edges, b2j_edges, j2b_edges):
    nb = branch_x.shape[0]
    nj = junction_x.shape[0]
    pk = lambda x: x.reshape(x.shape[0] // P, P * x.shape[1])
    pkc = lambda x: jnp.concatenate([x[j::P] for j in range(P)], axis=1)
    enc = lambda p, x: _tc_mlp([pkc(x)], [[0]], p, True, None)

    node_b = enc(params['enc_branch'], branch_x)
    node_j = enc(params['enc_junction'], junction_x)
    e_b2b = enc(params['enc_b2b'], b2b_eattr)
    e_j2j = enc(params['enc_j2j'], j2j_eattr)
    e_b2j = enc(params['enc_b2j'], b2j_eattr)
    e_j2b = enc(params['enc_b2j'], j2b_eattr)
    in_b = enc(params['enc_inlet'], inlet_branch)
    out_b = enc(params['enc_outlet'], outlet_branch)
    in_j = enc(params['enc_inlet'], inlet_junction)
    out_j = enc(params['enc_outlet'], outlet_junction)

    for i in range(3):
        tb = node_b.reshape(nb, 16)
        tj = node_j.reshape(nj, 16)
        gs, gd = _sc_gather_pair(tb, b2b_edges[0], tb, b2b_edges[1])
        e_b2b = _tc_mlp([e_b2b, pk(gs), pk(gd)], [[0], [1], [2]],
                        params['proc_b2b'][i], True, 0)
        gs, gd = _sc_gather_pair(tj, j2j_edges[0], tj, j2j_edges[1])
        e_j2j = _tc_mlp([e_j2j, pk(gs), pk(gd)], [[0], [1], [2]],
                        params['proc_j2j'][i], True, 0)
        gs, gd = _sc_gather_pair(tb, b2j_edges[0], tj, b2j_edges[1])
        e_b2j = _tc_mlp([e_b2j, pk(gs), pk(gd)], [[0], [1], [2]],
                        params['proc_b2j'][i], True, 0)
        gs, gd = _sc_gather_pair(tj, j2b_edges[0], tb, j2b_edges[1])
        e_j2b = _tc_mlp([e_j2b, pk(gs), pk(gd)], [[0], [1], [2]],
                        params['proc_j2b'][i], True, 0)
        pe_b = _sc_scatter2(e_b2b.reshape(-1, 16), b2b_edges[1],
                            e_j2b.reshape(-1, 16), j2b_edges[1], nb)
        pe_j = _sc_scatter2(e_j2j.reshape(-1, 16), j2j_edges[1],
                            e_b2j.reshape(-1, 16), b2j_edges[1], nj)
        pe_b = pe_b.reshape(NC, nb // P, P * 16)
        pe_j = pe_j.reshape(NC, nj // P, P * 16)
        node_b = _tc_mlp([node_b, pe_b, in_b, out_b], [[0], [1], [2], [3]],
                         params['proc_branch'][i], True, 0)
        node_j = _tc_mlp([node_j, pe_j, in_j, out_j], [[0], [1], [2], [3]],
                         params['proc_junction'][i], True, 0)

    h_b = _tc_mlp([node_b], [[0]], params['out_branch'], False, None)
    h_j = _tc_mlp([node_j], [[0]], params['out_junction'], False, None)
    return jnp.concatenate([h_b.reshape(nb, 2), h_j.reshape(nj, 2)], axis=0)
